# baseline (speedup 1.0000x reference)
import functools

import jax
import jax.numpy as jnp
from jax import lax
from jax.experimental import pallas as pl
from jax.experimental.pallas import tpu as pltpu
from jax.experimental.pallas import tpu_sc as plsc

L = 16


def _floor_f32(x):
    t = x.astype(jnp.int32)
    return t - jnp.where(t.astype(jnp.float32) > x, 1, 0)


def _sampler_body(nc, ns, ppw, ksub, h, w, c, log2_p,
                  map_hbm, pts_hbm, out_hbm,
                  tsc, pts_v, iw, rbuf, out_v, sbuf, tbuf,
                  gsems, tins, touts):
    cid = lax.axis_index("c")
    sid = lax.axis_index("s")
    wid = cid * ns + sid
    pbase = wid * ppw
    hw = h * w
    lanes = lax.iota(jnp.int32, L)
    lanes2 = lanes * 2

    nb_per_sc = ppw * ns // hw
    rows_per_slab = 2
    slabs_per_batch = h // (ns * rows_per_slab)
    nslab = nb_per_sc * slabs_per_batch
    spix = rows_per_slab * w

    def slab_ny(j):
        n = cid * nb_per_sc + lax.shift_right_logical(j, 2)
        y0 = rows_per_slab * sid + (j & (slabs_per_batch - 1)) * (
            ns * rows_per_slab)
        return n, y0

    def fire_in(j, slot):
        n, y0 = slab_ny(j)
        pltpu.async_copy(map_hbm.at[n, :, pl.ds(y0, rows_per_slab)],
                         sbuf[slot], tins[slot])

    def do_slab(j, slot):
        n, y0 = slab_ny(j)
        pltpu.make_async_copy(map_hbm.at[n, :, pl.ds(y0, rows_per_slab)],
                              sbuf[slot], tins[slot]).wait()
        sb = sbuf[slot]
        tb = tbuf[slot]

        @plsc.parallel_loop(0, spix)
        def _transpose(pix):
            ph = jnp.full((L,), lax.shift_right_logical(pix, 7), jnp.int32)
            pw = jnp.full((L,), pix & (w - 1), jnp.int32)
            for cb in range(c // (2 * L)):
                a = plsc.load_gather(sb, [lanes + cb * 2 * L, ph, pw])
                b = plsc.load_gather(sb, [lanes + (cb * 2 * L + L), ph, pw])
                tb[pix, pl.ds(cb * 2 * L, 2 * L)] = plsc.pack(
                    a, b, format=plsc.PackFormat.INTERLEAVED)

        pltpu.async_copy(tb, tsc.at[pl.ds(n * hw + y0 * w, spix)],
                         touts[slot])

    fire_in(0, 0)
    fire_in(1, 1)

    def trans2(j2, _):
        j = j2 * 2

        @pl.when(j >= 2)
        def _():
            pltpu.make_async_copy(tbuf[0], tsc.at[pl.ds(0, spix)],
                                  touts[0]).wait()

        do_slab(j, 0)

        @pl.when(j + 2 < nslab)
        def _():
            fire_in(j + 2, 0)

        @pl.when(j >= 1)
        def _():
            pltpu.make_async_copy(tbuf[1], tsc.at[pl.ds(0, spix)],
                                  touts[1]).wait()

        do_slab(j + 1, 1)

        @pl.when(j + 3 < nslab)
        def _():
            fire_in(j + 3, 1)

        return 0

    lax.fori_loop(0, nslab // 2, trans2, 0)
    pltpu.make_async_copy(tbuf[0], tsc.at[pl.ds(0, spix)], touts[0]).wait()
    pltpu.make_async_copy(tbuf[1], tsc.at[pl.ds(0, spix)], touts[1]).wait()
    plsc.subcore_barrier()

    pltpu.sync_copy(pts_hbm.at[pl.ds(2 * pbase, 2 * ppw)], pts_v)
    nsub = ppw // ksub

    def stage(j, slot):
        i_v = iw[slot][0]
        w_v = iw[slot][1]
        for g in range(ksub // L):
            off = j * ksub + g * L
            px = plsc.load_gather(pts_v, [lanes2 + 2 * off])
            py = plsc.load_gather(pts_v, [lanes2 + (2 * off + 1)])
            gx = (2.0 * px - 1.0) + 1.0
            gy = (2.0 * py - 1.0) + 1.0
            x = (gx * w - 1.0) * 0.5
            y = (gy * h - 1.0) * 0.5
            x0 = _floor_f32(x)
            y0 = _floor_f32(y)
            wx1 = x - x0.astype(jnp.float32)
            wx0 = 1.0 - wx1
            wy1 = y - y0.astype(jnp.float32)
            wy0 = 1.0 - wy1
            x1 = x0 + 1
            y1 = y0 + 1
            fx0 = jnp.where((x0 >= 0) & (x0 <= w - 1), wx0, 0.0)
            fx1 = jnp.where((x1 >= 0) & (x1 <= w - 1), wx1, 0.0)
            fy0 = jnp.where((y0 >= 0) & (y0 <= h - 1), wy0, 0.0)
            fy1 = jnp.where((y1 >= 0) & (y1 <= h - 1), wy1, 0.0)
            cx0 = jnp.minimum(jnp.maximum(x0, 0), w - 1)
            cx1 = jnp.minimum(jnp.maximum(x1, 0), w - 1)
            cy0 = jnp.minimum(jnp.maximum(y0, 0), h - 1)
            cy1 = jnp.minimum(jnp.maximum(y1, 0), h - 1)
            gp = pbase + off + lanes
            tb = gp & jnp.int32(~(2 ** log2_p - 1))
            row0 = tb + cy0 * w
            row1 = tb + cy1 * w
            sl = pl.ds(g * L, L)
            i_v[0][sl] = row0 + cx0
            i_v[1][sl] = row0 + cx1
            i_v[2][sl] = row1 + cx0
            i_v[3][sl] = row1 + cx1
            w_v[0][sl] = fy0 * fx0
            w_v[1][sl] = fy0 * fx1
            w_v[2][sl] = fy1 * fx0
            w_v[3][sl] = fy1 * fx1
        for k in range(4):
            pltpu.async_copy(tsc.at[i_v[k]], rbuf[slot][k], gsems[slot])

    def finish(j, slot):
        i_v = iw[slot][0]
        w_v = iw[slot][1]
        for k in range(4):
            pltpu.make_async_copy(tsc.at[i_v[k]], rbuf[slot][k],
                                  gsems[slot]).wait()

        @plsc.parallel_loop(0, ksub, unroll=2)
        def _combine(pt):
            idxv = jnp.full((L,), pt, jnp.int32)
            a00 = plsc.load_gather(w_v[0], [idxv])
            a01 = plsc.load_gather(w_v[1], [idxv])
            a10 = plsc.load_gather(w_v[2], [idxv])
            a11 = plsc.load_gather(w_v[3], [idxv])
            r00, r01, r10, r11 = rbuf[slot]
            fmt = plsc.PackFormat.INTERLEAVED
            for cb in range(c // (2 * L)):
                sl32 = pl.ds(cb * 2 * L, 2 * L)
                u0, u1 = plsc.unpack(r00[pt, sl32], format=fmt)
                v0, v1 = plsc.unpack(r01[pt, sl32], format=fmt)
                s0, s1 = plsc.unpack(r10[pt, sl32], format=fmt)
                t0, t1 = plsc.unpack(r11[pt, sl32], format=fmt)
                lo = (u0 * a00 + v0 * a01) + (s0 * a10 + t0 * a11)
                hi = (u1 * a00 + v1 * a01) + (s1 * a10 + t1 * a11)
                out_v[pt, pl.ds(cb * 2 * L, L)] = lo
                out_v[pt, pl.ds(cb * 2 * L + L, L)] = hi

        pltpu.sync_copy(out_v, out_hbm.at[pl.ds(pbase + j * ksub, ksub)])

    stage(0, 0)

    def sub2(j2, _):
        j = j2 * 2
        stage(j + 1, 1)
        finish(j, 0)

        @pl.when(j + 2 < nsub)
        def _():
            stage(j + 2, 0)

        finish(j + 1, 1)
        return 0

    lax.fori_loop(0, nsub // 2, sub2, 0)


def kernel(sample_map, sample_pts):
    n, c, h, w = sample_map.shape
    _, p, _ = sample_pts.shape
    np_total = n * p
    assert p & (p - 1) == 0 and h * w == p and w == 128
    log2_p = p.bit_length() - 1

    info = plsc.get_sparse_core_info()
    nc, ns = info.num_cores, info.num_subcores
    nw = nc * ns
    ppw = np_total // nw
    ksub = 64

    pts = sample_pts.reshape(-1)

    mesh = plsc.VectorSubcoreMesh(core_axis_name="c", subcore_axis_name="s")
    body = functools.partial(_sampler_body, nc, ns, ppw, ksub, h, w, c, log2_p)

    def wrapped(map_hbm, pts_hbm, out_hbm, tsc, pts_v,
                i000, i001, i010, i011, w000, w001, w010, w011,
                i100, i101, i110, i111, w100, w101, w110, w111,
                r000, r001, r010, r011, r100, r101, r110, r111,
                out_v, sb0, sb1, tb0, tb1,
                gsem0, gsem1, tin0, tin1, tout0, tout1):
        iw = (((i000, i001, i010, i011), (w000, w001, w010, w011)),
              ((i100, i101, i110, i111), (w100, w101, w110, w111)))
        rbuf = ((r000, r001, r010, r011), (r100, r101, r110, r111))
        body(map_hbm, pts_hbm, out_hbm, tsc, pts_v, iw, rbuf, out_v,
             (sb0, sb1), (tb0, tb1),
             (gsem0, gsem1), (tin0, tin1), (tout0, tout1))

    ivecs = [pltpu.VMEM((ksub,), jnp.int32)] * 4
    wvecs = [pltpu.VMEM((ksub,), jnp.float32)] * 4
    rvecs = [pltpu.VMEM((ksub, c), jnp.bfloat16)] * 4
    out = pl.kernel(
        wrapped,
        out_type=jax.ShapeDtypeStruct((np_total, c), jnp.float32),
        mesh=mesh,
        compiler_params=pltpu.CompilerParams(
            needs_layout_passes=False, use_tc_tiling_on_sc=False),
        scratch_types=(
            [pltpu.HBM((np_total, c), jnp.bfloat16)]
            + [pltpu.VMEM((2 * ppw,), jnp.float32)]
            + ivecs + wvecs + ivecs + wvecs + rvecs + rvecs
            + [pltpu.VMEM((ksub, c), jnp.float32)]
            + [pltpu.VMEM((c, 2, w), jnp.float32)] * 2
            + [pltpu.VMEM((2 * w, c), jnp.bfloat16)] * 2
            + [pltpu.SemaphoreType.DMA] * 6
        ),
    )(sample_map, pts)
    return out.reshape(n, p, c)
